# baseline (speedup 1.0000x reference)
import functools
import jax
import jax.numpy as jnp
from jax import lax
from jax.experimental import pallas as pl
from jax.experimental.pallas import tpu as pltpu
from jax.experimental.pallas import tpu_sc as plsc

D = 128
N_MAP = 10000
N_ACT = 1000
DIST_TH = 7.0
K = 64
M = 128
KC = 16
NPAD = 10240
NA_PAD = 1008
NW = 32
NPT = NPAD // NW


G = 4
NGRP = NPT // G


def _sc_build(cx_hbm, cy_hbm, ax_hbm, ay_hbm, actors_hbm,
              cnt_hbm, dxy_hbm, ag_hbm,
              axv, ayv, cxv, cyv, nbrA, nbrB, dxyA, dxyB, agA, agB, cntv,
              gsem0, gsem1, osem0, osem1):
    wid = lax.axis_index("s") * 2 + lax.axis_index("c")
    base = wid * NPT
    pltpu.sync_copy(ax_hbm, axv)
    pltpu.sync_copy(ay_hbm, ayv)
    pltpu.sync_copy(cx_hbm.at[pl.ds(base, NPT)], cxv)
    pltpu.sync_copy(cy_hbm.at[pl.ds(base, NPT)], cyv)
    lanes = lax.iota(jnp.int32, 16)
    zeros16 = jnp.zeros((16,), jnp.int32)
    th2 = DIST_TH * DIST_TH
    gsem = (gsem0, gsem1)
    osem = (osem0, osem1)
    nbrs = (nbrA, nbrB)
    dxys = (dxyA, dxyB)
    ags = (agA, agB)

    def group_body(it, _):
        for p in range(2):
            g = it * 2 + p
            @pl.when(it > 0)
            def _(p=p):
                pltpu.make_async_copy(
                    ag_hbm.at[pl.ds(0, G * K)], ags[p], osem[p]).wait()
                pltpu.make_async_copy(
                    dxy_hbm.at[pl.ds(0, G * K * 8)], dxys[p],
                    osem[p]).wait()

            for k in range(G):
                i = g * G + k
                isplat = jnp.zeros((16,), jnp.int32) + i
                cxi = plsc.load_gather(cxv, [isplat])
                cyi = plsc.load_gather(cyv, [isplat])
                for t in range(K // 16):
                    nbrs[p][pl.ds(k * K + t * 16, 16)] = zeros16

                def chunk(jc, cnt_n, cxi=cxi, cyi=cyi, p=p, k=k):
                    j0 = pl.multiple_of(jc * 16, 16)
                    dxl = cxi - axv[pl.ds(j0, 16)]
                    dyl = cyi - ayv[pl.ds(j0, 16)]
                    m = (dxl * dxl + dyl * dyl) <= th2
                    mi = m.astype(jnp.int32)
                    pos = cnt_n + plsc.cumsum(mi) - 1
                    ok = jnp.logical_and(m, pos < K)
                    posc = jnp.minimum(pos, K - 1)
                    plsc.store_scatter(nbrs[p], [k * K + posc],
                                       lanes + j0, mask=ok)
                    dbase = k * (K * 8)
                    plsc.store_scatter(dxys[p], [dbase + posc * 8],
                                       dxl, mask=ok)
                    plsc.store_scatter(dxys[p], [dbase + posc * 8 + 1],
                                       dyl, mask=ok)
                    return cnt_n + jnp.sum(mi)

                cnt_n = lax.fori_loop(0, NA_PAD // 16, chunk, jnp.int32(0))
                cntk = jnp.minimum(cnt_n, K)
                plsc.store_scatter(cntv, [jnp.zeros((16,), jnp.int32) + i],
                                   jnp.zeros((16,), jnp.int32) + cntk,
                                   mask=lanes == 0)

            descs = []
            for k in range(G):
                descs.append(pltpu.async_copy(
                    actors_hbm.at[nbrs[p].at[pl.ds(k * K, K)]],
                    ags[p].at[pl.ds(k * K, K)], gsem[p]))
            for d in descs:
                d.wait()

            row = pl.multiple_of((base + g * G) * K, 256)
            pltpu.async_copy(ags[p], ag_hbm.at[pl.ds(row, G * K)],
                             osem[p])
            off = pl.multiple_of((base + g * G) * K * 8, 2048)
            pltpu.async_copy(dxys[p], dxy_hbm.at[pl.ds(off, G * K * 8)],
                             osem[p])
        return 0

    lax.fori_loop(0, NGRP // 2, group_body, 0)
    for p in range(2):
        pltpu.make_async_copy(
            ag_hbm.at[pl.ds(0, G * K)], ags[p], osem[p]).wait()
        pltpu.make_async_copy(
            dxy_hbm.at[pl.ds(0, G * K * 8)], dxys[p], osem[p]).wait()
    pltpu.sync_copy(cntv, cnt_hbm.at[pl.ds(base, NPT)])


def _run_sc_build(cx, cy, ax, ay, actors):
    mesh = plsc.VectorSubcoreMesh(core_axis_name="c", subcore_axis_name="s")
    f = pl.kernel(
        _sc_build,
        out_type=(jax.ShapeDtypeStruct((NPAD,), jnp.int32),
                  jax.ShapeDtypeStruct((NPAD * K * 8,), jnp.float32),
                  jax.ShapeDtypeStruct((NPAD * K, D), jnp.float32)),
        mesh=mesh,
        compiler_params=pltpu.CompilerParams(needs_layout_passes=False),
        scratch_types=[
            pltpu.VMEM((NA_PAD,), jnp.float32),
            pltpu.VMEM((NA_PAD,), jnp.float32),
            pltpu.VMEM((NPT,), jnp.float32),
            pltpu.VMEM((NPT,), jnp.float32),
            pltpu.VMEM((G * K,), jnp.int32),
            pltpu.VMEM((G * K,), jnp.int32),
            pltpu.VMEM((G * K * 8,), jnp.float32),
            pltpu.VMEM((G * K * 8,), jnp.float32),
            pltpu.VMEM((G * K, D), jnp.float32),
            pltpu.VMEM((G * K, D), jnp.float32),
            pltpu.VMEM((NPT,), jnp.int32),
            pltpu.SemaphoreType.DMA,
            pltpu.SemaphoreType.DMA,
            pltpu.SemaphoreType.DMA,
            pltpu.SemaphoreType.DMA,
        ],
    )
    return f(cx, cy, ax, ay, actors)


def _gn(x, w, b):
    mu = jnp.mean(x, axis=-1, keepdims=True)
    var = jnp.mean((x - mu) ** 2, axis=-1, keepdims=True)
    return (x - mu) * jax.lax.rsqrt(var + 1e-5) * w + b


def _mlp_kernel(feat_ref, meta8_ref, cnt_ref, dxy_ref, ag_ref, vrow_ref,
                mwf_ref, mwm_ref,
                w1_0_ref, dw2_0_ref, qw_0_ref, wq_0_ref, ag_0_ref, wd_0_ref,
                wf_0_ref, cw2_0_ref, lin_0_ref,
                w1_1_ref, dw2_1_ref, qw_1_ref, wq_1_ref, ag_1_ref, wd_1_ref,
                wf_1_ref, cw2_1_ref, lin_1_ref,
                out_ref):
    v = vrow_ref[...]

    def row(i):
        return v[i][None, :]

    feat = feat_ref[...]
    meta8 = meta8_ref[...]
    cnt = cnt_ref[...]

    x = feat @ mwf_ref[...] + meta8 @ mwm_ref[...]
    x = jax.nn.relu(_gn(x, row(0), row(1)))

    iota3 = jax.lax.broadcasted_iota(jnp.int32, (M, KC, D), 1)

    blk = ((w1_0_ref, dw2_0_ref, qw_0_ref, wq_0_ref, ag_0_ref, wd_0_ref,
            wf_0_ref, cw2_0_ref, lin_0_ref),
           (w1_1_ref, dw2_1_ref, qw_1_ref, wq_1_ref, ag_1_ref, wd_1_ref,
            wf_1_ref, cw2_1_ref, lin_1_ref))

    for b in range(2):
        w1, dw2, qw, wq, agw, wd, wf, cw2, lin = blk[b]
        base = 2 + 11 * b
        q = jax.nn.relu(_gn(x @ qw[...], row(base + 3), row(base + 4)))
        qp = q @ wq[...]
        acc = x @ agw[...]

        w1m = w1[...]
        dw2m = dw2[...]
        wdm = wd[...]
        wfm = wf[...]
        cw2m = cw2[...]
        b1 = row(base + 0)
        dgw, dgb = row(base + 1), row(base + 2)
        cgw, cgb = row(base + 5), row(base + 6)

        for s in range(K // KC):
            dxy = dxy_ref[:, s * KC:(s + 1) * KC, :].reshape(M * KC, 8)
            agt = ag_ref[:, s * KC:(s + 1) * KC, :].reshape(M * KC, D)
            d1 = jax.nn.relu(dxy @ w1m + b1)
            d2 = jax.nn.relu(_gn(d1 @ dw2m, dgw, dgb))
            h = d2 @ wdm + agt @ wfm
            h = h.reshape(M, KC, D) + qp[:, None, :]
            h = jax.nn.relu(_gn(h, cgw[None], cgb[None]))
            c = h.reshape(M * KC, D) @ cw2m
            c = c.reshape(M, KC, D)
            valid = (iota3 + s * KC) < cnt[:, :, None]
            acc = acc + jnp.sum(jnp.where(valid, c, 0.0), axis=1)

        a = jax.nn.relu(_gn(acc, row(base + 7), row(base + 8)))
        a = _gn(a @ lin[...], row(base + 9), row(base + 10))
        x = jax.nn.relu(a + x)

    out_ref[...] = x


def _run_mlp(feat_p, meta8, cnt2, dxy8, ag, vrow, mats):
    grid = (NPAD // M,)
    bs_w = lambda shape: pl.BlockSpec(shape, lambda g: (0,) * len(shape))
    in_specs = [
        pl.BlockSpec((M, D), lambda g: (g, 0)),
        pl.BlockSpec((M, 8), lambda g: (g, 0)),
        pl.BlockSpec((M, 1), lambda g: (g, 0)),
        pl.BlockSpec((M, K, 8), lambda g: (g, 0, 0)),
        pl.BlockSpec((M, K, D), lambda g: (g, 0, 0)),
        bs_w(vrow.shape),
    ] + [bs_w(m.shape) for m in mats]
    return pl.pallas_call(
        _mlp_kernel,
        grid=grid,
        in_specs=in_specs,
        out_specs=pl.BlockSpec((M, D), lambda g: (g, 0)),
        out_shape=jax.ShapeDtypeStruct((NPAD, D), jnp.float32),
    )(feat_p, meta8, cnt2, dxy8, ag, vrow, *mats)


def kernel(feat, turn, control, intersect, ctrs, actors, actor_ctrs, idcs,
           actor_idcs, meta_w, meta_gw, meta_gb,
           b0_dist_w1, b0_dist_b1, b0_dist_w2, b0_dist_gw, b0_dist_gb,
           b0_query_w, b0_query_gw, b0_query_gb,
           b0_ctx_w1, b0_ctx_gw, b0_ctx_gb, b0_ctx_w2,
           b0_agt_w, b0_norm_w, b0_norm_b,
           b0_lin_w, b0_lin_gw, b0_lin_gb,
           b1_dist_w1, b1_dist_b1, b1_dist_w2, b1_dist_gw, b1_dist_gb,
           b1_query_w, b1_query_gw, b1_query_gb,
           b1_ctx_w1, b1_ctx_gw, b1_ctx_gb, b1_ctx_w2,
           b1_agt_w, b1_norm_w, b1_norm_b,
           b1_lin_w, b1_lin_gw, b1_lin_gb):
    pad = NPAD - N_MAP
    apad = NA_PAD - N_ACT
    cx = jnp.pad(ctrs[:, 0], (0, pad), constant_values=1e6)
    cy = jnp.pad(ctrs[:, 1], (0, pad), constant_values=1e6)
    ax = jnp.pad(actor_ctrs[:, 0], (0, apad), constant_values=-1e6)
    ay = jnp.pad(actor_ctrs[:, 1], (0, apad), constant_values=-1e6)
    cnt, dxy, ag = _run_sc_build(cx, cy, ax, ay, actors)
    cnt2 = cnt[:, None]
    dxy8 = dxy.reshape(NPAD, K, 8)
    ag = ag.reshape(NPAD, K, D)

    feat_p = jnp.pad(feat, ((0, pad), (0, 0)))
    meta = jnp.concatenate([turn, control[:, None], intersect[:, None]],
                           axis=1)
    meta8 = jnp.pad(meta, ((0, pad), (0, 4)))

    vrow = jnp.stack(
        [meta_gw, meta_gb,
         b0_dist_b1, b0_dist_gw, b0_dist_gb, b0_query_gw, b0_query_gb,
         b0_ctx_gw, b0_ctx_gb, b0_norm_w, b0_norm_b, b0_lin_gw, b0_lin_gb,
         b1_dist_b1, b1_dist_gw, b1_dist_gb, b1_query_gw, b1_query_gb,
         b1_ctx_gw, b1_ctx_gb, b1_norm_w, b1_norm_b, b1_lin_gw, b1_lin_gb])

    mwf = meta_w[:, :D].T
    mwm = jnp.pad(meta_w[:, D:].T, ((0, 4), (0, 0)))

    def blk_mats(dist_w1, dist_w2, query_w, ctx_w1, ctx_w2, agt_w, lin_w):
        w1 = jnp.pad(dist_w1.T, ((0, 6), (0, 0)))
        return (w1, dist_w2.T, query_w.T, ctx_w1[:, D:2 * D].T, agt_w.T,
                ctx_w1[:, :D].T, ctx_w1[:, 2 * D:].T, ctx_w2.T, lin_w.T)

    mats = ((mwf, mwm)
            + blk_mats(b0_dist_w1, b0_dist_w2, b0_query_w, b0_ctx_w1,
                       b0_ctx_w2, b0_agt_w, b0_lin_w)
            + blk_mats(b1_dist_w1, b1_dist_w2, b1_query_w, b1_ctx_w1,
                       b1_ctx_w2, b1_agt_w, b1_lin_w))

    out = _run_mlp(feat_p, meta8, cnt2, dxy8, ag, vrow, list(mats))
    return out[:N_MAP]


# speedup vs baseline: 7.6684x; 7.6684x over previous
import jax
import jax.numpy as jnp
from jax import lax
from jax.experimental import pallas as pl
from jax.experimental.pallas import tpu as pltpu
from jax.experimental.pallas import tpu_sc as plsc

D = 128
N_MAP = 10000
N_ACT = 1000
DIST_TH = 7.0
K = 64
M = 64
KC = 16
NPAD = 10240
NA_PAD = 1008
NA_OH = 1024
NW = 32
NPT = NPAD // NW
G = 4
BPT = NPT // M
NBLK = NPAD // M


def _sc_build(cx_hbm, cy_hbm, ax_hbm, ay_hbm,
              cnt_hbm, dxy_hbm, nbr_hbm,
              axv, ayv, cxv, cyv, nbrblk, dxyA, dxyB, cntv,
              osem0, osem1):
    wid = lax.axis_index("s") * 2 + lax.axis_index("c")
    base = wid * NPT
    pltpu.sync_copy(ax_hbm, axv)
    pltpu.sync_copy(ay_hbm, ayv)
    pltpu.sync_copy(cx_hbm.at[pl.ds(base, NPT)], cxv)
    pltpu.sync_copy(cy_hbm.at[pl.ds(base, NPT)], cyv)
    lanes = lax.iota(jnp.int32, 16)
    th2 = DIST_TH * DIST_TH
    osem = (osem0, osem1)
    dxys = (dxyA, dxyB)

    def blk_body(b, _):
        def grp2_body(it, _):
            for p in range(2):
                gg = it * 2 + p
                git = b * (M // G) + gg

                @pl.when(git > 1)
                def _(p=p):
                    pltpu.make_async_copy(
                        dxy_hbm.at[pl.ds(0, G * K * 8)], dxys[p],
                        osem[p]).wait()

                for k in range(G):
                    n_l = gg * G + k
                    i_t = b * M + n_l
                    isplat = jnp.zeros((16,), jnp.int32) + i_t
                    cxi = plsc.load_gather(cxv, [isplat])
                    cyi = plsc.load_gather(cyv, [isplat])

                    def chunk(jc, cnt_n, cxi=cxi, cyi=cyi, k=k, n_l=n_l,
                              p=p):
                        j0 = pl.multiple_of(jc * 16, 16)
                        dxl = cxi - axv[pl.ds(j0, 16)]
                        dyl = cyi - ayv[pl.ds(j0, 16)]
                        m = (dxl * dxl + dyl * dyl) <= th2
                        mi = m.astype(jnp.int32)
                        pos = cnt_n + plsc.cumsum(mi) - 1
                        ok = jnp.logical_and(m, pos < K)
                        posc = jnp.minimum(pos, K - 1)
                        oidx = ((posc >> 4) << 10) + n_l * 16 + (posc & 15)
                        plsc.store_scatter(nbrblk, [oidx],
                                           lanes + j0, mask=ok)
                        dbase = k * (K * 8)
                        plsc.store_scatter(dxys[p], [dbase + posc * 8],
                                           dxl, mask=ok)
                        plsc.store_scatter(dxys[p], [dbase + posc * 8 + 1],
                                           dyl, mask=ok)
                        return cnt_n + jnp.sum(mi)

                    cnt_n = lax.fori_loop(0, NA_PAD // 16, chunk,
                                          jnp.int32(0))
                    cntk = jnp.minimum(cnt_n, K)
                    plsc.store_scatter(
                        cntv, [jnp.zeros((16,), jnp.int32) + i_t],
                        jnp.zeros((16,), jnp.int32) + cntk,
                        mask=lanes == 0)

                off = pl.multiple_of((base + b * M + gg * G) * K * 8, 2048)
                pltpu.async_copy(dxys[p],
                                 dxy_hbm.at[pl.ds(off, G * K * 8)], osem[p])
            return 0

        lax.fori_loop(0, M // G // 2, grp2_body, 0)
        nrow = pl.multiple_of((wid * BPT + b) * (M * K), 4096)
        pltpu.sync_copy(nbrblk, nbr_hbm.at[pl.ds(nrow, M * K)])
        return 0

    lax.fori_loop(0, BPT, blk_body, 0)
    for p in range(2):
        pltpu.make_async_copy(
            dxy_hbm.at[pl.ds(0, G * K * 8)], dxys[p], osem[p]).wait()
    pltpu.sync_copy(cntv, cnt_hbm.at[pl.ds(base, NPT)])


def _run_sc_build(cx, cy, ax, ay):
    mesh = plsc.VectorSubcoreMesh(core_axis_name="c", subcore_axis_name="s")
    f = pl.kernel(
        _sc_build,
        out_type=(jax.ShapeDtypeStruct((NPAD,), jnp.int32),
                  jax.ShapeDtypeStruct((NPAD * K * 8,), jnp.float32),
                  jax.ShapeDtypeStruct((NPAD * K,), jnp.int32)),
        mesh=mesh,
        compiler_params=pltpu.CompilerParams(needs_layout_passes=False),
        scratch_types=[
            pltpu.VMEM((NA_PAD,), jnp.float32),
            pltpu.VMEM((NA_PAD,), jnp.float32),
            pltpu.VMEM((NPT,), jnp.float32),
            pltpu.VMEM((NPT,), jnp.float32),
            pltpu.VMEM((M * K,), jnp.int32),
            pltpu.VMEM((G * K * 8,), jnp.float32),
            pltpu.VMEM((G * K * 8,), jnp.float32),
            pltpu.VMEM((NPT,), jnp.int32),
            pltpu.SemaphoreType.DMA,
            pltpu.SemaphoreType.DMA,
        ],
    )
    return f(cx, cy, ax, ay)


def _gn(x, w, b):
    mu = jnp.mean(x, axis=-1, keepdims=True)
    var = jnp.mean((x - mu) ** 2, axis=-1, keepdims=True)
    return (x - mu) * jax.lax.rsqrt(var + 1e-5) * w + b


def _mlp_kernel(feat_ref, meta8_ref, cnt_ref, dxy_ref, nbr_ref, acts_ref,
                vrow_ref,
                mwf_ref, mwm_ref,
                w1_0_ref, dw2_0_ref, qw_0_ref, wq_0_ref, ag_0_ref, wd_0_ref,
                wf_0_ref, cw2_0_ref, lin_0_ref,
                w1_1_ref, dw2_1_ref, qw_1_ref, wq_1_ref, ag_1_ref, wd_1_ref,
                wf_1_ref, cw2_1_ref, lin_1_ref,
                out_ref):
    v = vrow_ref[...]

    def row(i):
        return v[i][None, :]

    feat = feat_ref[...]
    meta8 = meta8_ref[...]
    cnt = cnt_ref[...]

    x = feat @ mwf_ref[...] + meta8 @ mwm_ref[...]
    x = jax.nn.relu(_gn(x, row(0), row(1)))

    nbrflat = nbr_ref[...].reshape(M * K, 1)
    acts = acts_ref[...]
    iota_oh = lax.broadcasted_iota(jnp.int32, (M * KC, NA_OH), 1)
    agts = []
    for s in range(K // KC):
        col = nbrflat[s * M * KC:(s + 1) * M * KC]
        oh = (col == iota_oh).astype(jnp.bfloat16)
        agts.append(jnp.dot(oh, acts, preferred_element_type=jnp.float32))

    iota3 = jax.lax.broadcasted_iota(jnp.int32, (M, KC, D), 1)

    blk = ((w1_0_ref, dw2_0_ref, qw_0_ref, wq_0_ref, ag_0_ref, wd_0_ref,
            wf_0_ref, cw2_0_ref, lin_0_ref),
           (w1_1_ref, dw2_1_ref, qw_1_ref, wq_1_ref, ag_1_ref, wd_1_ref,
            wf_1_ref, cw2_1_ref, lin_1_ref))

    for b in range(2):
        w1, dw2, qw, wq, agw, wd, wf, cw2, lin = blk[b]
        base = 2 + 11 * b
        q = jax.nn.relu(_gn(x @ qw[...], row(base + 3), row(base + 4)))
        qp = q @ wq[...]
        acc = x @ agw[...]

        w1m = w1[...]
        dw2m = dw2[...]
        wdm = wd[...]
        wfm = wf[...]
        cw2m = cw2[...]
        b1 = row(base + 0)
        dgw, dgb = row(base + 1), row(base + 2)
        cgw, cgb = row(base + 5), row(base + 6)

        for s in range(K // KC):
            dxy = dxy_ref[:, s * KC:(s + 1) * KC, :].reshape(M * KC, 8)
            d1 = jax.nn.relu(dxy @ w1m + b1)
            d2 = jax.nn.relu(_gn(d1 @ dw2m, dgw, dgb))
            h = d2 @ wdm + agts[s] @ wfm
            h = h.reshape(M, KC, D) + qp[:, None, :]
            h = jax.nn.relu(_gn(h, cgw[None], cgb[None]))
            c = h.reshape(M * KC, D) @ cw2m
            c = c.reshape(M, KC, D)
            valid = (iota3 + s * KC) < cnt[:, :, None]
            acc = acc + jnp.sum(jnp.where(valid, c, 0.0), axis=1)

        a = jax.nn.relu(_gn(acc, row(base + 7), row(base + 8)))
        a = _gn(a @ lin[...], row(base + 9), row(base + 10))
        x = jax.nn.relu(a + x)

    out_ref[...] = x


def _run_mlp(feat_p, meta8, cnt2, dxy8, nbr3, acts_bf, vrow, mats):
    grid = (NBLK,)
    bs_w = lambda shape: pl.BlockSpec(shape, lambda g: (0,) * len(shape))
    in_specs = [
        pl.BlockSpec((M, D), lambda g: (g, 0)),
        pl.BlockSpec((M, 8), lambda g: (g, 0)),
        pl.BlockSpec((M, 1), lambda g: (g, 0)),
        pl.BlockSpec((M, K, 8), lambda g: (g, 0, 0)),
        pl.BlockSpec((1, M * K, 1), lambda g: (g, 0, 0)),
        bs_w(acts_bf.shape),
        bs_w(vrow.shape),
    ] + [bs_w(m.shape) for m in mats]
    return pl.pallas_call(
        _mlp_kernel,
        grid=grid,
        in_specs=in_specs,
        out_specs=pl.BlockSpec((M, D), lambda g: (g, 0)),
        out_shape=jax.ShapeDtypeStruct((NPAD, D), jnp.float32),
    )(feat_p, meta8, cnt2, dxy8, nbr3, acts_bf, vrow, *mats)


def kernel(feat, turn, control, intersect, ctrs, actors, actor_ctrs, idcs,
           actor_idcs, meta_w, meta_gw, meta_gb,
           b0_dist_w1, b0_dist_b1, b0_dist_w2, b0_dist_gw, b0_dist_gb,
           b0_query_w, b0_query_gw, b0_query_gb,
           b0_ctx_w1, b0_ctx_gw, b0_ctx_gb, b0_ctx_w2,
           b0_agt_w, b0_norm_w, b0_norm_b,
           b0_lin_w, b0_lin_gw, b0_lin_gb,
           b1_dist_w1, b1_dist_b1, b1_dist_w2, b1_dist_gw, b1_dist_gb,
           b1_query_w, b1_query_gw, b1_query_gb,
           b1_ctx_w1, b1_ctx_gw, b1_ctx_gb, b1_ctx_w2,
           b1_agt_w, b1_norm_w, b1_norm_b,
           b1_lin_w, b1_lin_gw, b1_lin_gb):
    pad = NPAD - N_MAP
    apad = NA_PAD - N_ACT
    cx = jnp.pad(ctrs[:, 0], (0, pad), constant_values=1e6)
    cy = jnp.pad(ctrs[:, 1], (0, pad), constant_values=1e6)
    ax = jnp.pad(actor_ctrs[:, 0], (0, apad), constant_values=-1e6)
    ay = jnp.pad(actor_ctrs[:, 1], (0, apad), constant_values=-1e6)
    cnt, dxy, nbr = _run_sc_build(cx, cy, ax, ay)
    cnt2 = cnt[:, None]
    dxy8 = dxy.reshape(NPAD, K, 8)
    nbr3 = nbr.reshape(NBLK, M * K, 1)

    feat_p = jnp.pad(feat, ((0, pad), (0, 0)))
    meta = jnp.concatenate([turn, control[:, None], intersect[:, None]],
                           axis=1)
    meta8 = jnp.pad(meta, ((0, pad), (0, 4)))
    acts_bf = jnp.pad(actors, ((0, NA_OH - N_ACT), (0, 0))).astype(
        jnp.bfloat16)

    vrow = jnp.stack(
        [meta_gw, meta_gb,
         b0_dist_b1, b0_dist_gw, b0_dist_gb, b0_query_gw, b0_query_gb,
         b0_ctx_gw, b0_ctx_gb, b0_norm_w, b0_norm_b, b0_lin_gw, b0_lin_gb,
         b1_dist_b1, b1_dist_gw, b1_dist_gb, b1_query_gw, b1_query_gb,
         b1_ctx_gw, b1_ctx_gb, b1_norm_w, b1_norm_b, b1_lin_gw, b1_lin_gb])

    mwf = meta_w[:, :D].T
    mwm = jnp.pad(meta_w[:, D:].T, ((0, 4), (0, 0)))

    def blk_mats(dist_w1, dist_w2, query_w, ctx_w1, ctx_w2, agt_w, lin_w):
        w1 = jnp.pad(dist_w1.T, ((0, 6), (0, 0)))
        return (w1, dist_w2.T, query_w.T, ctx_w1[:, D:2 * D].T, agt_w.T,
                ctx_w1[:, :D].T, ctx_w1[:, 2 * D:].T, ctx_w2.T, lin_w.T)

    mats = ((mwf, mwm)
            + blk_mats(b0_dist_w1, b0_dist_w2, b0_query_w, b0_ctx_w1,
                       b0_ctx_w2, b0_agt_w, b0_lin_w)
            + blk_mats(b1_dist_w1, b1_dist_w2, b1_query_w, b1_ctx_w1,
                       b1_ctx_w2, b1_agt_w, b1_lin_w))

    out = _run_mlp(feat_p, meta8, cnt2, dxy8, nbr3, acts_bf, vrow,
                   list(mats))
    return out[:N_MAP]


# speedup vs baseline: 9.3640x; 1.2211x over previous
import jax
import jax.numpy as jnp
from jax import lax
from jax.experimental import pallas as pl
from jax.experimental.pallas import tpu as pltpu
from jax.experimental.pallas import tpu_sc as plsc

D = 128
N_MAP = 10000
N_ACT = 1000
DIST_TH = 7.0
K = 64
M = 64
KC = 16
NPAD = 10240
NA_PAD = 1008
NA_OH = 1024
NW = 32
NPT = NPAD // NW
G = 4
BPT = NPT // M
NBLK = NPAD // M


def _sc_build(cx_hbm, cy_hbm, ax_hbm, ay_hbm,
              cnt_hbm, dxy_hbm, nbr_hbm,
              axv, ayv, cxv, cyv, nbrblk, dxyA, dxyB, cntv,
              osem0, osem1):
    wid = lax.axis_index("s") * 2 + lax.axis_index("c")
    base = wid * NPT
    pltpu.sync_copy(ax_hbm, axv)
    pltpu.sync_copy(ay_hbm, ayv)
    pltpu.sync_copy(cx_hbm.at[pl.ds(base, NPT)], cxv)
    pltpu.sync_copy(cy_hbm.at[pl.ds(base, NPT)], cyv)
    lanes = lax.iota(jnp.int32, 16)
    th2 = DIST_TH * DIST_TH
    osem = (osem0, osem1)
    dxys = (dxyA, dxyB)

    def blk_body(b, _):
        def grp2_body(it, _):
            for p in range(2):
                gg = it * 2 + p
                git = b * (M // G) + gg

                @pl.when(git > 1)
                def _(p=p):
                    pltpu.make_async_copy(
                        dxy_hbm.at[pl.ds(0, G * K * 8)], dxys[p],
                        osem[p]).wait()

                for k in range(G):
                    n_l = gg * G + k
                    i_t = b * M + n_l
                    isplat = jnp.zeros((16,), jnp.int32) + i_t
                    cxi = plsc.load_gather(cxv, [isplat])
                    cyi = plsc.load_gather(cyv, [isplat])

                    def chunk(jc, cnt_n, cxi=cxi, cyi=cyi, k=k, n_l=n_l,
                              p=p):
                        j0 = pl.multiple_of(jc * 16, 16)
                        dxl = cxi - axv[pl.ds(j0, 16)]
                        dyl = cyi - ayv[pl.ds(j0, 16)]
                        m = (dxl * dxl + dyl * dyl) <= th2
                        mi = m.astype(jnp.int32)
                        pos = cnt_n + plsc.cumsum(mi) - 1
                        ok = jnp.logical_and(m, pos < K)
                        posc = jnp.minimum(pos, K - 1)
                        oidx = ((posc >> 4) << 10) + n_l * 16 + (posc & 15)
                        plsc.store_scatter(nbrblk, [oidx],
                                           lanes + j0, mask=ok)
                        dbase = k * (K * 8)
                        plsc.store_scatter(dxys[p], [dbase + posc * 8],
                                           dxl, mask=ok)
                        plsc.store_scatter(dxys[p], [dbase + posc * 8 + 1],
                                           dyl, mask=ok)
                        return cnt_n + jnp.sum(mi)

                    cnt_n = lax.fori_loop(0, NA_PAD // 16, chunk,
                                          jnp.int32(0))
                    cntk = jnp.minimum(cnt_n, K)
                    plsc.store_scatter(
                        cntv, [jnp.zeros((16,), jnp.int32) + i_t],
                        jnp.zeros((16,), jnp.int32) + cntk,
                        mask=lanes == 0)

                off = pl.multiple_of((base + b * M + gg * G) * K * 8, 2048)
                pltpu.async_copy(dxys[p],
                                 dxy_hbm.at[pl.ds(off, G * K * 8)], osem[p])
            return 0

        lax.fori_loop(0, M // G // 2, grp2_body, 0)
        nrow = pl.multiple_of((wid * BPT + b) * (M * K), 4096)
        pltpu.sync_copy(nbrblk, nbr_hbm.at[pl.ds(nrow, M * K)])
        return 0

    lax.fori_loop(0, BPT, blk_body, 0)
    for p in range(2):
        pltpu.make_async_copy(
            dxy_hbm.at[pl.ds(0, G * K * 8)], dxys[p], osem[p]).wait()
    pltpu.sync_copy(cntv, cnt_hbm.at[pl.ds(base, NPT)])


def _run_sc_build(cx, cy, ax, ay):
    mesh = plsc.VectorSubcoreMesh(core_axis_name="c", subcore_axis_name="s")
    f = pl.kernel(
        _sc_build,
        out_type=(jax.ShapeDtypeStruct((NPAD,), jnp.int32),
                  jax.ShapeDtypeStruct((NPAD * K * 8,), jnp.float32),
                  jax.ShapeDtypeStruct((NPAD * K,), jnp.int32)),
        mesh=mesh,
        compiler_params=pltpu.CompilerParams(needs_layout_passes=False),
        scratch_types=[
            pltpu.VMEM((NA_PAD,), jnp.float32),
            pltpu.VMEM((NA_PAD,), jnp.float32),
            pltpu.VMEM((NPT,), jnp.float32),
            pltpu.VMEM((NPT,), jnp.float32),
            pltpu.VMEM((M * K,), jnp.int32),
            pltpu.VMEM((G * K * 8,), jnp.float32),
            pltpu.VMEM((G * K * 8,), jnp.float32),
            pltpu.VMEM((NPT,), jnp.int32),
            pltpu.SemaphoreType.DMA,
            pltpu.SemaphoreType.DMA,
        ],
    )
    return f(cx, cy, ax, ay)


def _gn(x, w, b):
    mu = jnp.mean(x, axis=-1, keepdims=True)
    var = jnp.mean((x - mu) ** 2, axis=-1, keepdims=True)
    return (x - mu) * jax.lax.rsqrt(var + 1e-5) * w + b


def _gn_mx(z, w, b, selA, selB):
    s8 = z @ selA
    t8 = (z * z) @ selA
    inv8 = jax.lax.rsqrt(t8 - s8 * s8 + 1e-5)
    mub = s8 @ selB
    invb = inv8 @ selB
    return (z - mub) * invb * w + b


def _mlp_kernel(feat_ref, meta8_ref, cnt_ref, dxy_ref, nbr_ref, acts_ref,
                vrow_ref,
                mwf_ref, mwm_ref,
                w1_0_ref, dw2_0_ref, qw_0_ref, wq_0_ref, ag_0_ref, wd_0_ref,
                wf_0_ref, cw2_0_ref, lin_0_ref,
                w1_1_ref, dw2_1_ref, qw_1_ref, wq_1_ref, ag_1_ref, wd_1_ref,
                wf_1_ref, cw2_1_ref, lin_1_ref,
                out_ref):
    v = vrow_ref[...]

    def row(i):
        return v[i][None, :]

    feat = feat_ref[...]
    meta8 = meta8_ref[...]
    cnt = cnt_ref[...]

    ci8 = lax.broadcasted_iota(jnp.int32, (D, 8), 1)
    selA = jnp.where(ci8 == 0, 1.0 / D, 0.0)
    ri8 = lax.broadcasted_iota(jnp.int32, (8, D), 0)
    selB = jnp.where(ri8 == 0, 1.0, 0.0)
    r0 = lax.broadcasted_iota(jnp.int32, (M * KC, M), 0) >> 4
    r1 = lax.broadcasted_iota(jnp.int32, (M * KC, M), 1)
    rsel = jnp.where(r0 == r1, 1.0, 0.0)

    def gn(zz, ww, bb):
        return _gn(zz, ww, bb)

    x = feat @ mwf_ref[...] + meta8 @ mwm_ref[...]
    x = jax.nn.relu(gn(x, row(0), row(1)))

    maxcnt = jnp.max(cnt)
    nbrflat = nbr_ref[...].reshape(M * K, 1)
    acts = acts_ref[...]
    iota_oh = lax.broadcasted_iota(jnp.int32, (M * KC, NA_OH), 1)

    def oh_dot(col):
        oh = (col == iota_oh).astype(jnp.bfloat16)
        return jnp.dot(oh, acts,
                       preferred_element_type=jnp.float32
                       ).astype(jnp.bfloat16)

    agts = []
    for s in range(K // KC):
        col = nbrflat[s * M * KC:(s + 1) * M * KC]
        if s == 0:
            agts.append(oh_dot(col))
        else:
            agts.append(lax.cond(
                s * KC < maxcnt, oh_dot,
                lambda c: jnp.zeros((M * KC, D), jnp.bfloat16), col))

    iota3 = jax.lax.broadcasted_iota(jnp.int32, (M, KC, D), 1)

    blk = ((w1_0_ref, dw2_0_ref, qw_0_ref, wq_0_ref, ag_0_ref, wd_0_ref,
            wf_0_ref, cw2_0_ref, lin_0_ref),
           (w1_1_ref, dw2_1_ref, qw_1_ref, wq_1_ref, ag_1_ref, wd_1_ref,
            wf_1_ref, cw2_1_ref, lin_1_ref))

    for b in range(2):
        w1, dw2, qw, wq, agw, wd, wf, cw2, lin = blk[b]
        base = 2 + 11 * b
        q = jax.nn.relu(gn(x @ qw[...], row(base + 3), row(base + 4)))
        qp = q @ wq[...]
        acc = x @ agw[...]

        w1m = w1[...]
        dw2m = dw2[...]
        wdm = wd[...]
        wfm = wf[...]
        cw2m = cw2[...]
        b1 = row(base + 0)
        dgw, dgb = row(base + 1), row(base + 2)
        cgw, cgb = row(base + 5), row(base + 6)

        qp_rep = rsel @ qp

        for s in range(K // KC):
            dxy = dxy_ref[:, s * KC:(s + 1) * KC, :].reshape(M * KC, 8)

            def chunk_fn(dxy, agt, qp_rep, s=s):
                d1 = jax.nn.relu(dxy @ w1m + b1).astype(jnp.bfloat16)
                d1m = jnp.dot(d1, dw2m, preferred_element_type=jnp.float32)
                d2 = jax.nn.relu(gn(d1m, dgw, dgb)).astype(jnp.bfloat16)
                h = (jnp.dot(d2, wdm, preferred_element_type=jnp.float32)
                     + jnp.dot(agt, wfm,
                               preferred_element_type=jnp.float32)
                     + qp_rep)
                h = jax.nn.relu(gn(h, cgw, cgb)).astype(jnp.bfloat16)
                c = jnp.dot(h, cw2m, preferred_element_type=jnp.float32)
                c = c.reshape(M, KC, D)
                valid = (iota3 + s * KC) < cnt[:, :, None]
                return jnp.sum(jnp.where(valid, c, 0.0), axis=1)

            if s == 0:
                acc = acc + chunk_fn(dxy, agts[s], qp_rep)
            else:
                acc = acc + lax.cond(
                    s * KC < maxcnt, chunk_fn,
                    lambda d, a, q: jnp.zeros((M, D), jnp.float32),
                    dxy, agts[s], qp_rep)

        a = jax.nn.relu(gn(acc, row(base + 7), row(base + 8)))
        a = gn(a @ lin[...], row(base + 9), row(base + 10))
        x = jax.nn.relu(a + x)

    out_ref[...] = x


def _run_mlp(feat_p, meta8, cnt2, dxy8, nbr3, acts_bf, vrow, mats):
    grid = (NBLK,)
    bs_w = lambda shape: pl.BlockSpec(shape, lambda g: (0,) * len(shape))
    in_specs = [
        pl.BlockSpec((M, D), lambda g: (g, 0)),
        pl.BlockSpec((M, 8), lambda g: (g, 0)),
        pl.BlockSpec((M, 1), lambda g: (g, 0)),
        pl.BlockSpec((M, K, 8), lambda g: (g, 0, 0)),
        pl.BlockSpec((1, M * K, 1), lambda g: (g, 0, 0)),
        bs_w(acts_bf.shape),
        bs_w(vrow.shape),
    ] + [bs_w(m.shape) for m in mats]
    return pl.pallas_call(
        _mlp_kernel,
        grid=grid,
        in_specs=in_specs,
        out_specs=pl.BlockSpec((M, D), lambda g: (g, 0)),
        out_shape=jax.ShapeDtypeStruct((NPAD, D), jnp.float32),
    )(feat_p, meta8, cnt2, dxy8, nbr3, acts_bf, vrow, *mats)


def kernel(feat, turn, control, intersect, ctrs, actors, actor_ctrs, idcs,
           actor_idcs, meta_w, meta_gw, meta_gb,
           b0_dist_w1, b0_dist_b1, b0_dist_w2, b0_dist_gw, b0_dist_gb,
           b0_query_w, b0_query_gw, b0_query_gb,
           b0_ctx_w1, b0_ctx_gw, b0_ctx_gb, b0_ctx_w2,
           b0_agt_w, b0_norm_w, b0_norm_b,
           b0_lin_w, b0_lin_gw, b0_lin_gb,
           b1_dist_w1, b1_dist_b1, b1_dist_w2, b1_dist_gw, b1_dist_gb,
           b1_query_w, b1_query_gw, b1_query_gb,
           b1_ctx_w1, b1_ctx_gw, b1_ctx_gb, b1_ctx_w2,
           b1_agt_w, b1_norm_w, b1_norm_b,
           b1_lin_w, b1_lin_gw, b1_lin_gb):
    pad = NPAD - N_MAP
    apad = NA_PAD - N_ACT
    cx = jnp.pad(ctrs[:, 0], (0, pad), constant_values=1e6)
    cy = jnp.pad(ctrs[:, 1], (0, pad), constant_values=1e6)
    ax = jnp.pad(actor_ctrs[:, 0], (0, apad), constant_values=-1e6)
    ay = jnp.pad(actor_ctrs[:, 1], (0, apad), constant_values=-1e6)
    cnt, dxy, nbr = _run_sc_build(cx, cy, ax, ay)
    cnt2 = cnt[:, None]
    dxy8 = dxy.reshape(NPAD, K, 8)
    nbr3 = nbr.reshape(NBLK, M * K, 1)

    feat_p = jnp.pad(feat, ((0, pad), (0, 0)))
    meta = jnp.concatenate([turn, control[:, None], intersect[:, None]],
                           axis=1)
    meta8 = jnp.pad(meta, ((0, pad), (0, 4)))
    acts_bf = jnp.pad(actors, ((0, NA_OH - N_ACT), (0, 0))).astype(
        jnp.bfloat16)

    vrow = jnp.stack(
        [meta_gw, meta_gb,
         b0_dist_b1, b0_dist_gw, b0_dist_gb, b0_query_gw, b0_query_gb,
         b0_ctx_gw, b0_ctx_gb, b0_norm_w, b0_norm_b, b0_lin_gw, b0_lin_gb,
         b1_dist_b1, b1_dist_gw, b1_dist_gb, b1_query_gw, b1_query_gb,
         b1_ctx_gw, b1_ctx_gb, b1_norm_w, b1_norm_b, b1_lin_gw, b1_lin_gb])

    mwf = meta_w[:, :D].T
    mwm = jnp.pad(meta_w[:, D:].T, ((0, 4), (0, 0)))

    def blk_mats(dist_w1, dist_w2, query_w, ctx_w1, ctx_w2, agt_w, lin_w):
        w1 = jnp.pad(dist_w1.T, ((0, 6), (0, 0)))
        bf = jnp.bfloat16
        return (w1, dist_w2.T.astype(bf), query_w.T,
                ctx_w1[:, D:2 * D].T, agt_w.T,
                ctx_w1[:, :D].T.astype(bf), ctx_w1[:, 2 * D:].T.astype(bf),
                ctx_w2.T.astype(bf), lin_w.T)

    mats = ((mwf, mwm)
            + blk_mats(b0_dist_w1, b0_dist_w2, b0_query_w, b0_ctx_w1,
                       b0_ctx_w2, b0_agt_w, b0_lin_w)
            + blk_mats(b1_dist_w1, b1_dist_w2, b1_query_w, b1_ctx_w1,
                       b1_ctx_w2, b1_agt_w, b1_lin_w))

    out = _run_mlp(feat_p, meta8, cnt2, dxy8, nbr3, acts_bf, vrow,
                   list(mats))
    return out[:N_MAP]


# speedup vs baseline: 10.9137x; 1.1655x over previous
import jax
import jax.numpy as jnp
from jax import lax
from jax.experimental import pallas as pl
from jax.experimental.pallas import tpu as pltpu
from jax.experimental.pallas import tpu_sc as plsc

D = 128
N_MAP = 10000
N_ACT = 1000
DIST_TH = 7.0
K = 64
M = 64
KC = 16
NPAD = 10240
NA_PAD = 1024
NA_OH = 1024
NW = 32
NPT = NPAD // NW
G = 4
BPT = NPT // M
NBLK = NPAD // M


def _sc_build(cx_hbm, cy_hbm, ax_hbm, ay_hbm,
              cnt_hbm, dxy_hbm, nbr_hbm,
              axv, ayv, cxv, cyv, nbrblk, dxyA, dxyB, cntv,
              cminv, cmaxv, osem0, osem1):
    wid = lax.axis_index("s") * 2 + lax.axis_index("c")
    base = wid * NPT
    pltpu.sync_copy(ax_hbm, axv)
    pltpu.sync_copy(ay_hbm, ayv)
    pltpu.sync_copy(cx_hbm.at[pl.ds(base, NPT)], cxv)
    pltpu.sync_copy(cy_hbm.at[pl.ds(base, NPT)], cyv)
    lanes = lax.iota(jnp.int32, 16)
    th2 = DIST_TH * DIST_TH
    osem = (osem0, osem1)
    dxys = (dxyA, dxyB)

    for t in range(NA_PAD // 256):
        cminv[pl.ds(t * 16, 16)] = plsc.load_gather(
            axv, [lanes * 16 + t * 256])
        cmaxv[pl.ds(t * 16, 16)] = plsc.load_gather(
            axv, [lanes * 16 + t * 256 + 15])

    def blk_body(b, _):
        def grp2_body(it, _):
            for p in range(2):
                gg = it * 2 + p
                git = b * (M // G) + gg

                @pl.when(git > 1)
                def _(p=p):
                    pltpu.make_async_copy(
                        dxy_hbm.at[pl.ds(0, G * K * 8)], dxys[p],
                        osem[p]).wait()

                for k in range(G):
                    n_l = gg * G + k
                    i_t = b * M + n_l
                    isplat = jnp.zeros((16,), jnp.int32) + i_t
                    cxi = plsc.load_gather(cxv, [isplat])
                    cyi = plsc.load_gather(cyv, [isplat])
                    lo_s = jnp.int32(0)
                    hi_s = jnp.int32(0)
                    for t in range(NA_PAD // 256):
                        cmx = cmaxv[pl.ds(t * 16, 16)]
                        cmn = cminv[pl.ds(t * 16, 16)]
                        lo_s = lo_s + jnp.sum(
                            (cmx < cxi - DIST_TH).astype(jnp.int32))
                        hi_s = hi_s + jnp.sum(
                            (cmn <= cxi + DIST_TH).astype(jnp.int32))

                    def chunk(jc, cnt_n, cxi=cxi, cyi=cyi, k=k, n_l=n_l,
                              p=p):
                        j0 = pl.multiple_of(jc * 16, 16)
                        dxl = cxi - axv[pl.ds(j0, 16)]
                        dyl = cyi - ayv[pl.ds(j0, 16)]
                        m = (dxl * dxl + dyl * dyl) <= th2
                        mi = m.astype(jnp.int32)
                        pos = cnt_n + plsc.cumsum(mi) - 1
                        ok = jnp.logical_and(m, pos < K)
                        posc = jnp.minimum(pos, K - 1)
                        oidx = ((posc >> 4) << 10) + n_l * 16 + (posc & 15)
                        plsc.store_scatter(nbrblk, [oidx],
                                           lanes + j0, mask=ok)
                        dbase = k * (K * 8)
                        plsc.store_scatter(dxys[p], [dbase + posc * 8],
                                           dxl, mask=ok)
                        plsc.store_scatter(dxys[p], [dbase + posc * 8 + 1],
                                           dyl, mask=ok)
                        return cnt_n + jnp.sum(mi)

                    cnt_n = lax.fori_loop(lo_s, hi_s, chunk,
                                          jnp.int32(0))
                    cntk = jnp.minimum(cnt_n, K)
                    plsc.store_scatter(
                        cntv, [jnp.zeros((16,), jnp.int32) + i_t],
                        jnp.zeros((16,), jnp.int32) + cntk,
                        mask=lanes == 0)

                off = pl.multiple_of((base + b * M + gg * G) * K * 8, 2048)
                pltpu.async_copy(dxys[p],
                                 dxy_hbm.at[pl.ds(off, G * K * 8)], osem[p])
            return 0

        lax.fori_loop(0, M // G // 2, grp2_body, 0)
        nrow = pl.multiple_of((wid * BPT + b) * (M * K), 4096)
        pltpu.sync_copy(nbrblk, nbr_hbm.at[pl.ds(nrow, M * K)])
        return 0

    lax.fori_loop(0, BPT, blk_body, 0)
    for p in range(2):
        pltpu.make_async_copy(
            dxy_hbm.at[pl.ds(0, G * K * 8)], dxys[p], osem[p]).wait()
    pltpu.sync_copy(cntv, cnt_hbm.at[pl.ds(base, NPT)])


def _run_sc_build(cx, cy, ax, ay):
    mesh = plsc.VectorSubcoreMesh(core_axis_name="c", subcore_axis_name="s")
    f = pl.kernel(
        _sc_build,
        out_type=(jax.ShapeDtypeStruct((NPAD,), jnp.int32),
                  jax.ShapeDtypeStruct((NPAD * K * 8,), jnp.float32),
                  jax.ShapeDtypeStruct((NPAD * K,), jnp.int32)),
        mesh=mesh,
        compiler_params=pltpu.CompilerParams(needs_layout_passes=False),
        scratch_types=[
            pltpu.VMEM((NA_PAD,), jnp.float32),
            pltpu.VMEM((NA_PAD,), jnp.float32),
            pltpu.VMEM((NPT,), jnp.float32),
            pltpu.VMEM((NPT,), jnp.float32),
            pltpu.VMEM((M * K,), jnp.int32),
            pltpu.VMEM((G * K * 8,), jnp.float32),
            pltpu.VMEM((G * K * 8,), jnp.float32),
            pltpu.VMEM((NPT,), jnp.int32),
            pltpu.VMEM((NA_PAD // 16,), jnp.float32),
            pltpu.VMEM((NA_PAD // 16,), jnp.float32),
            pltpu.SemaphoreType.DMA,
            pltpu.SemaphoreType.DMA,
        ],
    )
    return f(cx, cy, ax, ay)


def _gn(x, w, b):
    mu = jnp.mean(x, axis=-1, keepdims=True)
    var = jnp.mean((x - mu) ** 2, axis=-1, keepdims=True)
    return (x - mu) * jax.lax.rsqrt(var + 1e-5) * w + b


def _gn_mx(z, w, b, selA, selB):
    s8 = z @ selA
    t8 = (z * z) @ selA
    inv8 = jax.lax.rsqrt(t8 - s8 * s8 + 1e-5)
    mub = s8 @ selB
    invb = inv8 @ selB
    return (z - mub) * invb * w + b


def _mlp_kernel(feat_ref, meta8_ref, cnt_ref, dxy_ref, nbr_ref, acts_ref,
                vrow_ref,
                mwf_ref, mwm_ref,
                w1_0_ref, dw2_0_ref, qw_0_ref, wq_0_ref, ag_0_ref, wd_0_ref,
                wf_0_ref, cw2_0_ref, lin_0_ref,
                w1_1_ref, dw2_1_ref, qw_1_ref, wq_1_ref, ag_1_ref, wd_1_ref,
                wf_1_ref, cw2_1_ref, lin_1_ref,
                out_ref):
    v = vrow_ref[...]

    def row(i):
        return v[i][None, :]

    feat = feat_ref[...]
    meta8 = meta8_ref[...]
    cnt = cnt_ref[...]

    ci8 = lax.broadcasted_iota(jnp.int32, (D, 8), 1)
    selA = jnp.where(ci8 == 0, 1.0 / D, 0.0)
    ri8 = lax.broadcasted_iota(jnp.int32, (8, D), 0)
    selB = jnp.where(ri8 == 0, 1.0, 0.0)
    r0 = lax.broadcasted_iota(jnp.int32, (M * KC, M), 0) >> 4
    r1 = lax.broadcasted_iota(jnp.int32, (M * KC, M), 1)
    rsel = jnp.where(r0 == r1, 1.0, 0.0)

    def gn(zz, ww, bb):
        return _gn(zz, ww, bb)

    x = feat @ mwf_ref[...] + meta8 @ mwm_ref[...]
    x = jax.nn.relu(gn(x, row(0), row(1)))

    maxcnt = jnp.max(cnt)
    nbrflat = nbr_ref[...].reshape(M * K, 1)
    acts = acts_ref[...]
    iota_oh = lax.broadcasted_iota(jnp.int32, (M * KC, NA_OH), 1)

    def oh_dot(col):
        oh = (col == iota_oh).astype(jnp.bfloat16)
        return jnp.dot(oh, acts,
                       preferred_element_type=jnp.float32
                       ).astype(jnp.bfloat16)

    agts = []
    for s in range(K // KC):
        col = nbrflat[s * M * KC:(s + 1) * M * KC]
        if s == 0:
            agts.append(oh_dot(col))
        else:
            agts.append(lax.cond(
                s * KC < maxcnt, oh_dot,
                lambda c: jnp.zeros((M * KC, D), jnp.bfloat16), col))

    iota3 = jax.lax.broadcasted_iota(jnp.int32, (M, KC, D), 1)

    blk = ((w1_0_ref, dw2_0_ref, qw_0_ref, wq_0_ref, ag_0_ref, wd_0_ref,
            wf_0_ref, cw2_0_ref, lin_0_ref),
           (w1_1_ref, dw2_1_ref, qw_1_ref, wq_1_ref, ag_1_ref, wd_1_ref,
            wf_1_ref, cw2_1_ref, lin_1_ref))

    for b in range(2):
        w1, dw2, qw, wq, agw, wd, wf, cw2, lin = blk[b]
        base = 2 + 11 * b
        q = jax.nn.relu(gn(x @ qw[...], row(base + 3), row(base + 4)))
        qp = q @ wq[...]
        acc = x @ agw[...]

        w1m = w1[...]
        dw2m = dw2[...]
        wdm = wd[...]
        wfm = wf[...]
        cw2m = cw2[...]
        b1 = row(base + 0)
        dgw, dgb = row(base + 1), row(base + 2)
        cgw, cgb = row(base + 5), row(base + 6)

        qp_rep = rsel @ qp

        for s in range(K // KC):
            dxy = dxy_ref[:, s * KC:(s + 1) * KC, :].reshape(M * KC, 8)

            def chunk_fn(dxy, agt, qp_rep, s=s):
                d1 = jax.nn.relu(dxy @ w1m + b1).astype(jnp.bfloat16)
                d1m = jnp.dot(d1, dw2m, preferred_element_type=jnp.float32)
                d2 = jax.nn.relu(gn(d1m, dgw, dgb)).astype(jnp.bfloat16)
                h = (jnp.dot(d2, wdm, preferred_element_type=jnp.float32)
                     + jnp.dot(agt, wfm,
                               preferred_element_type=jnp.float32)
                     + qp_rep)
                h = jax.nn.relu(gn(h, cgw, cgb)).astype(jnp.bfloat16)
                c = jnp.dot(h, cw2m, preferred_element_type=jnp.float32)
                c = c.reshape(M, KC, D)
                valid = (iota3 + s * KC) < cnt[:, :, None]
                return jnp.sum(jnp.where(valid, c, 0.0), axis=1)

            if s == 0:
                acc = acc + chunk_fn(dxy, agts[s], qp_rep)
            else:
                acc = acc + lax.cond(
                    s * KC < maxcnt, chunk_fn,
                    lambda d, a, q: jnp.zeros((M, D), jnp.float32),
                    dxy, agts[s], qp_rep)

        a = jax.nn.relu(gn(acc, row(base + 7), row(base + 8)))
        a = gn(a @ lin[...], row(base + 9), row(base + 10))
        x = jax.nn.relu(a + x)

    out_ref[...] = x


def _run_mlp(feat_p, meta8, cnt2, dxy8, nbr3, acts_bf, vrow, mats):
    grid = (NBLK,)
    bs_w = lambda shape: pl.BlockSpec(shape, lambda g: (0,) * len(shape))
    in_specs = [
        pl.BlockSpec((M, D), lambda g: (g, 0)),
        pl.BlockSpec((M, 8), lambda g: (g, 0)),
        pl.BlockSpec((M, 1), lambda g: (g, 0)),
        pl.BlockSpec((M, K, 8), lambda g: (g, 0, 0)),
        pl.BlockSpec((1, M * K, 1), lambda g: (g, 0, 0)),
        bs_w(acts_bf.shape),
        bs_w(vrow.shape),
    ] + [bs_w(m.shape) for m in mats]
    return pl.pallas_call(
        _mlp_kernel,
        grid=grid,
        in_specs=in_specs,
        out_specs=pl.BlockSpec((M, D), lambda g: (g, 0)),
        out_shape=jax.ShapeDtypeStruct((NPAD, D), jnp.float32),
    )(feat_p, meta8, cnt2, dxy8, nbr3, acts_bf, vrow, *mats)


def kernel(feat, turn, control, intersect, ctrs, actors, actor_ctrs, idcs,
           actor_idcs, meta_w, meta_gw, meta_gb,
           b0_dist_w1, b0_dist_b1, b0_dist_w2, b0_dist_gw, b0_dist_gb,
           b0_query_w, b0_query_gw, b0_query_gb,
           b0_ctx_w1, b0_ctx_gw, b0_ctx_gb, b0_ctx_w2,
           b0_agt_w, b0_norm_w, b0_norm_b,
           b0_lin_w, b0_lin_gw, b0_lin_gb,
           b1_dist_w1, b1_dist_b1, b1_dist_w2, b1_dist_gw, b1_dist_gb,
           b1_query_w, b1_query_gw, b1_query_gb,
           b1_ctx_w1, b1_ctx_gw, b1_ctx_gb, b1_ctx_w2,
           b1_agt_w, b1_norm_w, b1_norm_b,
           b1_lin_w, b1_lin_gw, b1_lin_gb):
    order = jnp.argsort(actor_ctrs[:, 0])
    actor_ctrs = actor_ctrs[order]
    actors = actors[order]
    pad = NPAD - N_MAP
    apad = NA_PAD - N_ACT
    cx = jnp.pad(ctrs[:, 0], (0, pad), constant_values=1e6)
    cy = jnp.pad(ctrs[:, 1], (0, pad), constant_values=1e6)
    ax = jnp.pad(actor_ctrs[:, 0], (0, apad), constant_values=2e6)
    ay = jnp.pad(actor_ctrs[:, 1], (0, apad), constant_values=2e6)
    cnt, dxy, nbr = _run_sc_build(cx, cy, ax, ay)
    cnt2 = cnt[:, None]
    dxy8 = dxy.reshape(NPAD, K, 8)
    nbr3 = nbr.reshape(NBLK, M * K, 1)

    feat_p = jnp.pad(feat, ((0, pad), (0, 0)))
    meta = jnp.concatenate([turn, control[:, None], intersect[:, None]],
                           axis=1)
    meta8 = jnp.pad(meta, ((0, pad), (0, 4)))
    acts_bf = jnp.pad(actors, ((0, NA_OH - N_ACT), (0, 0))).astype(
        jnp.bfloat16)

    vrow = jnp.stack(
        [meta_gw, meta_gb,
         b0_dist_b1, b0_dist_gw, b0_dist_gb, b0_query_gw, b0_query_gb,
         b0_ctx_gw, b0_ctx_gb, b0_norm_w, b0_norm_b, b0_lin_gw, b0_lin_gb,
         b1_dist_b1, b1_dist_gw, b1_dist_gb, b1_query_gw, b1_query_gb,
         b1_ctx_gw, b1_ctx_gb, b1_norm_w, b1_norm_b, b1_lin_gw, b1_lin_gb])

    mwf = meta_w[:, :D].T
    mwm = jnp.pad(meta_w[:, D:].T, ((0, 4), (0, 0)))

    def blk_mats(dist_w1, dist_w2, query_w, ctx_w1, ctx_w2, agt_w, lin_w):
        w1 = jnp.pad(dist_w1.T, ((0, 6), (0, 0)))
        bf = jnp.bfloat16
        return (w1, dist_w2.T.astype(bf), query_w.T,
                ctx_w1[:, D:2 * D].T, agt_w.T,
                ctx_w1[:, :D].T.astype(bf), ctx_w1[:, 2 * D:].T.astype(bf),
                ctx_w2.T.astype(bf), lin_w.T)

    mats = ((mwf, mwm)
            + blk_mats(b0_dist_w1, b0_dist_w2, b0_query_w, b0_ctx_w1,
                       b0_ctx_w2, b0_agt_w, b0_lin_w)
            + blk_mats(b1_dist_w1, b1_dist_w2, b1_query_w, b1_ctx_w1,
                       b1_ctx_w2, b1_agt_w, b1_lin_w))

    out = _run_mlp(feat_p, meta8, cnt2, dxy8, nbr3, acts_bf, vrow,
                   list(mats))
    return out[:N_MAP]


# speedup vs baseline: 11.9304x; 1.0932x over previous
import jax
import jax.numpy as jnp
from jax import lax
from jax.experimental import pallas as pl
from jax.experimental.pallas import tpu as pltpu
from jax.experimental.pallas import tpu_sc as plsc

D = 128
N_MAP = 10000
N_ACT = 1000
DIST_TH = 7.0
K = 64
M = 64
KC = 32
NPAD = 10240
NA_PAD = 1024
NA_OH = 1024
NW = 32
NPT = NPAD // NW
G = 4
BPT = NPT // M
NBLK = NPAD // M


def _sc_build(cx_hbm, cy_hbm, ax_hbm, ay_hbm,
              cnt_hbm, dxy_hbm, nbr_hbm,
              axv, ayv, cxv, cyv, nbrblk, dxyA, dxyB, cntv,
              cminv, cmaxv, osem0, osem1):
    wid = lax.axis_index("s") * 2 + lax.axis_index("c")
    base = wid * NPT
    pltpu.sync_copy(ax_hbm, axv)
    pltpu.sync_copy(ay_hbm, ayv)
    pltpu.sync_copy(cx_hbm.at[pl.ds(base, NPT)], cxv)
    pltpu.sync_copy(cy_hbm.at[pl.ds(base, NPT)], cyv)
    lanes = lax.iota(jnp.int32, 16)
    th2 = DIST_TH * DIST_TH
    osem = (osem0, osem1)
    dxys = (dxyA, dxyB)

    for t in range(NA_PAD // 256):
        cminv[pl.ds(t * 16, 16)] = plsc.load_gather(
            axv, [lanes * 16 + t * 256])
        cmaxv[pl.ds(t * 16, 16)] = plsc.load_gather(
            axv, [lanes * 16 + t * 256 + 15])

    def blk_body(b, _):
        def grp2_body(it, _):
            for p in range(2):
                gg = it * 2 + p
                git = b * (M // G) + gg

                @pl.when(git > 1)
                def _(p=p):
                    pltpu.make_async_copy(
                        dxy_hbm.at[pl.ds(0, G * K * 8)], dxys[p],
                        osem[p]).wait()

                for k in range(G):
                    n_l = gg * G + k
                    i_t = b * M + n_l
                    isplat = jnp.zeros((16,), jnp.int32) + i_t
                    cxi = plsc.load_gather(cxv, [isplat])
                    cyi = plsc.load_gather(cyv, [isplat])
                    lo_s = jnp.int32(0)
                    hi_s = jnp.int32(0)
                    for t in range(NA_PAD // 256):
                        cmx = cmaxv[pl.ds(t * 16, 16)]
                        cmn = cminv[pl.ds(t * 16, 16)]
                        lo_s = lo_s + jnp.sum(
                            (cmx < cxi - DIST_TH).astype(jnp.int32))
                        hi_s = hi_s + jnp.sum(
                            (cmn <= cxi + DIST_TH).astype(jnp.int32))

                    def chunk(jc, cnt_n, cxi=cxi, cyi=cyi, k=k, n_l=n_l,
                              p=p):
                        j0 = pl.multiple_of(jc * 16, 16)
                        dxl = cxi - axv[pl.ds(j0, 16)]
                        dyl = cyi - ayv[pl.ds(j0, 16)]
                        m = (dxl * dxl + dyl * dyl) <= th2
                        mi = m.astype(jnp.int32)
                        pos = cnt_n + plsc.cumsum(mi) - 1
                        ok = jnp.logical_and(m, pos < K)
                        posc = jnp.minimum(pos, K - 1)
                        oidx = (((posc // KC) * (M * KC)) + n_l * KC
                                + (posc % KC))
                        plsc.store_scatter(nbrblk, [oidx],
                                           lanes + j0, mask=ok)
                        dbase = k * (K * 8)
                        plsc.store_scatter(dxys[p], [dbase + posc * 8],
                                           dxl, mask=ok)
                        plsc.store_scatter(dxys[p], [dbase + posc * 8 + 1],
                                           dyl, mask=ok)
                        return cnt_n + jnp.sum(mi)

                    cnt_n = lax.fori_loop(lo_s, hi_s, chunk,
                                          jnp.int32(0))
                    cntk = jnp.minimum(cnt_n, K)
                    plsc.store_scatter(
                        cntv, [jnp.zeros((16,), jnp.int32) + i_t],
                        jnp.zeros((16,), jnp.int32) + cntk,
                        mask=lanes == 0)

                off = pl.multiple_of((base + b * M + gg * G) * K * 8, 2048)
                pltpu.async_copy(dxys[p],
                                 dxy_hbm.at[pl.ds(off, G * K * 8)], osem[p])
            return 0

        lax.fori_loop(0, M // G // 2, grp2_body, 0)
        nrow = pl.multiple_of((wid * BPT + b) * (M * K), 4096)
        pltpu.sync_copy(nbrblk, nbr_hbm.at[pl.ds(nrow, M * K)])
        return 0

    lax.fori_loop(0, BPT, blk_body, 0)
    for p in range(2):
        pltpu.make_async_copy(
            dxy_hbm.at[pl.ds(0, G * K * 8)], dxys[p], osem[p]).wait()
    pltpu.sync_copy(cntv, cnt_hbm.at[pl.ds(base, NPT)])


def _run_sc_build(cx, cy, ax, ay):
    mesh = plsc.VectorSubcoreMesh(core_axis_name="c", subcore_axis_name="s")
    f = pl.kernel(
        _sc_build,
        out_type=(jax.ShapeDtypeStruct((NPAD,), jnp.int32),
                  jax.ShapeDtypeStruct((NPAD * K * 8,), jnp.float32),
                  jax.ShapeDtypeStruct((NPAD * K,), jnp.int32)),
        mesh=mesh,
        compiler_params=pltpu.CompilerParams(needs_layout_passes=False),
        scratch_types=[
            pltpu.VMEM((NA_PAD,), jnp.float32),
            pltpu.VMEM((NA_PAD,), jnp.float32),
            pltpu.VMEM((NPT,), jnp.float32),
            pltpu.VMEM((NPT,), jnp.float32),
            pltpu.VMEM((M * K,), jnp.int32),
            pltpu.VMEM((G * K * 8,), jnp.float32),
            pltpu.VMEM((G * K * 8,), jnp.float32),
            pltpu.VMEM((NPT,), jnp.int32),
            pltpu.VMEM((NA_PAD // 16,), jnp.float32),
            pltpu.VMEM((NA_PAD // 16,), jnp.float32),
            pltpu.SemaphoreType.DMA,
            pltpu.SemaphoreType.DMA,
        ],
    )
    return f(cx, cy, ax, ay)


def _gn(x, w, b):
    mu = jnp.mean(x, axis=-1, keepdims=True)
    var = jnp.mean((x - mu) ** 2, axis=-1, keepdims=True)
    return (x - mu) * jax.lax.rsqrt(var + 1e-5) * w + b


def _gn_mx(z, w, b, selA, selB):
    s8 = z @ selA
    t8 = (z * z) @ selA
    inv8 = jax.lax.rsqrt(t8 - s8 * s8 + 1e-5)
    mub = s8 @ selB
    invb = inv8 @ selB
    return (z - mub) * invb * w + b


def _mlp_kernel(feat_ref, meta8_ref, cnt_ref, dxy_ref, nbr_ref, acts_ref,
                vrow_ref,
                mwf_ref, mwm_ref,
                w1_0_ref, dw2_0_ref, qw_0_ref, wq_0_ref, ag_0_ref, wd_0_ref,
                wf_0_ref, cw2_0_ref, lin_0_ref,
                w1_1_ref, dw2_1_ref, qw_1_ref, wq_1_ref, ag_1_ref, wd_1_ref,
                wf_1_ref, cw2_1_ref, lin_1_ref,
                out_ref):
    v = vrow_ref[...]

    def row(i):
        return v[i][None, :]

    feat = feat_ref[...]
    meta8 = meta8_ref[...]
    cnt = cnt_ref[...]

    ci8 = lax.broadcasted_iota(jnp.int32, (D, 8), 1)
    selA = jnp.where(ci8 == 0, 1.0 / D, 0.0)
    ri8 = lax.broadcasted_iota(jnp.int32, (8, D), 0)
    selB = jnp.where(ri8 == 0, 1.0, 0.0)
    r0 = lax.broadcasted_iota(jnp.int32, (M * KC, M), 0) // KC
    r1 = lax.broadcasted_iota(jnp.int32, (M * KC, M), 1)
    rsel = jnp.where(r0 == r1, 1.0, 0.0)

    def gn(zz, ww, bb):
        return _gn(zz, ww, bb)

    x = feat @ mwf_ref[...] + meta8 @ mwm_ref[...]
    x = jax.nn.relu(gn(x, row(0), row(1)))

    maxcnt = jnp.max(cnt)
    nbrflat = nbr_ref[...].reshape(M * K, 1)
    acts = acts_ref[...]
    iota_oh = lax.broadcasted_iota(jnp.int32, (M * KC, NA_OH), 1)

    def oh_dot(col):
        oh = (col == iota_oh).astype(jnp.bfloat16)
        return jnp.dot(oh, acts,
                       preferred_element_type=jnp.float32
                       ).astype(jnp.bfloat16)

    agts = []
    for s in range(K // KC):
        col = nbrflat[s * M * KC:(s + 1) * M * KC]
        if s == 0:
            agts.append(oh_dot(col))
        else:
            agts.append(lax.cond(
                s * KC < maxcnt, oh_dot,
                lambda c: jnp.zeros((M * KC, D), jnp.bfloat16), col))

    iota3 = jax.lax.broadcasted_iota(jnp.int32, (M, KC, D), 1)

    blk = ((w1_0_ref, dw2_0_ref, qw_0_ref, wq_0_ref, ag_0_ref, wd_0_ref,
            wf_0_ref, cw2_0_ref, lin_0_ref),
           (w1_1_ref, dw2_1_ref, qw_1_ref, wq_1_ref, ag_1_ref, wd_1_ref,
            wf_1_ref, cw2_1_ref, lin_1_ref))

    for b in range(2):
        w1, dw2, qw, wq, agw, wd, wf, cw2, lin = blk[b]
        base = 2 + 11 * b
        q = jax.nn.relu(gn(x @ qw[...], row(base + 3), row(base + 4)))
        qp = q @ wq[...]
        acc = x @ agw[...]

        w1m = w1[...]
        dw2m = dw2[...]
        wdm = wd[...]
        wfm = wf[...]
        cw2m = cw2[...]
        b1 = row(base + 0)
        dgw, dgb = row(base + 1), row(base + 2)
        cgw, cgb = row(base + 5), row(base + 6)

        qp_rep = rsel @ qp

        for s in range(K // KC):
            dxy = dxy_ref[:, s * KC:(s + 1) * KC, :].reshape(M * KC, 8)

            def chunk_fn(dxy, agt, qp_rep, s=s):
                d1 = jax.nn.relu(dxy @ w1m + b1).astype(jnp.bfloat16)
                d1m = jnp.dot(d1, dw2m, preferred_element_type=jnp.float32)
                d2 = jax.nn.relu(gn(d1m, dgw, dgb)).astype(jnp.bfloat16)
                h = (jnp.dot(d2, wdm, preferred_element_type=jnp.float32)
                     + jnp.dot(agt, wfm,
                               preferred_element_type=jnp.float32)
                     + qp_rep)
                h = jax.nn.relu(gn(h, cgw, cgb)).astype(jnp.bfloat16)
                c = jnp.dot(h, cw2m, preferred_element_type=jnp.float32)
                c = c.reshape(M, KC, D)
                valid = (iota3 + s * KC) < cnt[:, :, None]
                return jnp.sum(jnp.where(valid, c, 0.0), axis=1)

            if s == 0:
                acc = acc + chunk_fn(dxy, agts[s], qp_rep)
            else:
                acc = acc + lax.cond(
                    s * KC < maxcnt, chunk_fn,
                    lambda d, a, q: jnp.zeros((M, D), jnp.float32),
                    dxy, agts[s], qp_rep)

        a = jax.nn.relu(gn(acc, row(base + 7), row(base + 8)))
        a = gn(a @ lin[...], row(base + 9), row(base + 10))
        x = jax.nn.relu(a + x)

    out_ref[...] = x


def _run_mlp(feat_p, meta8, cnt2, dxy8, nbr3, acts_bf, vrow, mats):
    grid = (NBLK,)
    bs_w = lambda shape: pl.BlockSpec(shape, lambda g: (0,) * len(shape))
    in_specs = [
        pl.BlockSpec((M, D), lambda g: (g, 0)),
        pl.BlockSpec((M, 8), lambda g: (g, 0)),
        pl.BlockSpec((M, 1), lambda g: (g, 0)),
        pl.BlockSpec((M, K, 8), lambda g: (g, 0, 0)),
        pl.BlockSpec((1, M * K, 1), lambda g: (g, 0, 0)),
        bs_w(acts_bf.shape),
        bs_w(vrow.shape),
    ] + [bs_w(m.shape) for m in mats]
    return pl.pallas_call(
        _mlp_kernel,
        grid=grid,
        in_specs=in_specs,
        out_specs=pl.BlockSpec((M, D), lambda g: (g, 0)),
        out_shape=jax.ShapeDtypeStruct((NPAD, D), jnp.float32),
    )(feat_p, meta8, cnt2, dxy8, nbr3, acts_bf, vrow, *mats)


def kernel(feat, turn, control, intersect, ctrs, actors, actor_ctrs, idcs,
           actor_idcs, meta_w, meta_gw, meta_gb,
           b0_dist_w1, b0_dist_b1, b0_dist_w2, b0_dist_gw, b0_dist_gb,
           b0_query_w, b0_query_gw, b0_query_gb,
           b0_ctx_w1, b0_ctx_gw, b0_ctx_gb, b0_ctx_w2,
           b0_agt_w, b0_norm_w, b0_norm_b,
           b0_lin_w, b0_lin_gw, b0_lin_gb,
           b1_dist_w1, b1_dist_b1, b1_dist_w2, b1_dist_gw, b1_dist_gb,
           b1_query_w, b1_query_gw, b1_query_gb,
           b1_ctx_w1, b1_ctx_gw, b1_ctx_gb, b1_ctx_w2,
           b1_agt_w, b1_norm_w, b1_norm_b,
           b1_lin_w, b1_lin_gw, b1_lin_gb):
    order = jnp.argsort(actor_ctrs[:, 0])
    actor_ctrs = actor_ctrs[order]
    actors = actors[order]
    pad = NPAD - N_MAP
    apad = NA_PAD - N_ACT
    cx = jnp.pad(ctrs[:, 0], (0, pad), constant_values=1e6)
    cy = jnp.pad(ctrs[:, 1], (0, pad), constant_values=1e6)
    ax = jnp.pad(actor_ctrs[:, 0], (0, apad), constant_values=2e6)
    ay = jnp.pad(actor_ctrs[:, 1], (0, apad), constant_values=2e6)
    cnt, dxy, nbr = _run_sc_build(cx, cy, ax, ay)
    cnt2 = cnt[:, None]
    dxy8 = dxy.reshape(NPAD, K, 8)
    nbr3 = nbr.reshape(NBLK, M * K, 1)

    feat_p = jnp.pad(feat, ((0, pad), (0, 0)))
    meta = jnp.concatenate([turn, control[:, None], intersect[:, None]],
                           axis=1)
    meta8 = jnp.pad(meta, ((0, pad), (0, 4)))
    acts_bf = jnp.pad(actors, ((0, NA_OH - N_ACT), (0, 0))).astype(
        jnp.bfloat16)

    vrow = jnp.stack(
        [meta_gw, meta_gb,
         b0_dist_b1, b0_dist_gw, b0_dist_gb, b0_query_gw, b0_query_gb,
         b0_ctx_gw, b0_ctx_gb, b0_norm_w, b0_norm_b, b0_lin_gw, b0_lin_gb,
         b1_dist_b1, b1_dist_gw, b1_dist_gb, b1_query_gw, b1_query_gb,
         b1_ctx_gw, b1_ctx_gb, b1_norm_w, b1_norm_b, b1_lin_gw, b1_lin_gb])

    mwf = meta_w[:, :D].T
    mwm = jnp.pad(meta_w[:, D:].T, ((0, 4), (0, 0)))

    def blk_mats(dist_w1, dist_w2, query_w, ctx_w1, ctx_w2, agt_w, lin_w):
        w1 = jnp.pad(dist_w1.T, ((0, 6), (0, 0)))
        bf = jnp.bfloat16
        return (w1, dist_w2.T.astype(bf), query_w.T,
                ctx_w1[:, D:2 * D].T, agt_w.T,
                ctx_w1[:, :D].T.astype(bf), ctx_w1[:, 2 * D:].T.astype(bf),
                ctx_w2.T.astype(bf), lin_w.T)

    mats = ((mwf, mwm)
            + blk_mats(b0_dist_w1, b0_dist_w2, b0_query_w, b0_ctx_w1,
                       b0_ctx_w2, b0_agt_w, b0_lin_w)
            + blk_mats(b1_dist_w1, b1_dist_w2, b1_query_w, b1_ctx_w1,
                       b1_ctx_w2, b1_agt_w, b1_lin_w))

    out = _run_mlp(feat_p, meta8, cnt2, dxy8, nbr3, acts_bf, vrow,
                   list(mats))
    return out[:N_MAP]
